# baseline (speedup 1.0000x reference)
import functools

import numpy as np

import jax
import jax.numpy as jnp
from jax import lax
from jax.experimental import pallas as pl
from jax.experimental.pallas import tpu as pltpu
from jax.experimental.pallas import tpu_sc as plsc

_LANES = 16


def _argmax_body(x_ref, i_ref, o_ref):
    x = x_ref[...]
    m = jnp.max(x, axis=2, keepdims=True)
    ii = i_ref[...]
    cand = jnp.where(x == m, ii, jnp.float32(x_ref.shape[2]))
    o_ref[...] = jnp.min(cand, axis=2)


def _finish_body(a_ref, b_ref, o_ref, inv_n):
    t = jnp.sum(a_ref[...], axis=0) + jnp.sum(b_ref[...], axis=0)
    m = jnp.max(t, axis=1, keepdims=True)
    o_ref[...] = jnp.sum(m, axis=0, keepdims=True) * inv_n


def _make_hist(nh, n_clus, n_clas, tgt_off):
    info = plsc.get_sparse_core_info()
    ns = info.num_subcores
    chunk = nh // ns
    unroll = 4
    assert chunk % (_LANES * unroll) == 0 and chunk % 8 == 0
    mesh = plsc.VectorSubcoreMesh(
        core_axis_name="c", subcore_axis_name="s", num_cores=1)

    @functools.partial(
        pl.kernel,
        out_type=jax.ShapeDtypeStruct((ns, n_clus, n_clas), jnp.float32),
        mesh=mesh,
        scratch_types=[
            pltpu.VMEM((chunk,), jnp.float32),
            pltpu.VMEM((chunk,), jnp.int32),
            pltpu.VMEM((n_clus, n_clas), jnp.float32),
        ],
        compiler_params=pltpu.CompilerParams(needs_layout_passes=False),
    )
    def hist(clus_hbm, tgt_hbm, out_hbm, clus_v, tgt_v, tab):
        wid = lax.axis_index("s")
        base = wid * chunk
        pltpu.sync_copy(clus_hbm.at[pl.ds(base, chunk)], clus_v)
        pltpu.sync_copy(tgt_hbm.at[pl.ds(tgt_off + base, chunk)], tgt_v)

        zeros = jnp.zeros((_LANES,), jnp.float32)

        def zero_step(i, carry):
            for j in range(n_clas // _LANES):
                tab[i, pl.ds(j * _LANES, _LANES)] = zeros
            return carry

        lax.fori_loop(0, n_clus, zero_step, 0)

        ones = jnp.ones((_LANES,), jnp.float32)

        def step(i, carry):
            for j in range(unroll):
                off = (i * unroll + j) * _LANES
                cv = clus_v[pl.ds(off, _LANES)].astype(jnp.int32)
                tv = tgt_v[pl.ds(off, _LANES)]
                plsc.addupdate_scatter(tab, [cv, tv], ones)
            return carry

        lax.fori_loop(0, chunk // (_LANES * unroll), step, 0)
        pltpu.sync_copy(tab, out_hbm.at[wid])

    return hist, ns


def kernel(inputs, targets):
    n, n_clus = inputs.shape
    n_clas = 128
    row_blk = 64
    parts = [3 * n // 4, n // 4]

    x3 = inputs.reshape(n // n_clus, n_clus, n_clus)
    lane_idx = jnp.asarray(
        np.arange(n_clus, dtype=np.float32).reshape(1, 1, n_clus))

    tabs = []
    off = 0
    for nh in parts:
        steps = (nh // n_clus) // row_blk
        base_step = (off // n_clus) // row_blk
        clus_h = pl.pallas_call(
            _argmax_body,
            grid=(steps,),
            in_specs=[
                pl.BlockSpec((row_blk, n_clus, n_clus),
                             lambda i, b=base_step: (b + i, 0, 0)),
                pl.BlockSpec((1, 1, n_clus), lambda i: (0, 0, 0)),
            ],
            out_specs=pl.BlockSpec((row_blk, n_clus), lambda i: (i, 0)),
            out_shape=jax.ShapeDtypeStruct((nh // n_clus, n_clus),
                                           jnp.float32),
        )(x3, lane_idx)
        hist, _ = _make_hist(nh, n_clus, n_clas, off)
        tabs.append(hist(clus_h.reshape(nh), targets))
        off += nh

    out = pl.pallas_call(
        functools.partial(_finish_body, inv_n=1.0 / n),
        out_shape=jax.ShapeDtypeStruct((1, 1), jnp.float32),
    )(tabs[0], tabs[1])
    return out.reshape(1)


# speedup vs baseline: 1.0668x; 1.0668x over previous
import functools

import numpy as np

import jax
import jax.numpy as jnp
from jax import lax
from jax.experimental import pallas as pl
from jax.experimental.pallas import tpu as pltpu
from jax.experimental.pallas import tpu_sc as plsc

_LANES = 16


def _argmax_body(x_ref, i_ref, o_ref):
    x = x_ref[...]
    m = jnp.max(x, axis=2, keepdims=True)
    ii = i_ref[...]
    cand = jnp.where(x == m, ii, jnp.float32(x_ref.shape[2]))
    o_ref[...] = jnp.min(cand, axis=2)


def _finish_body(t_ref, o_ref, inv_n):
    t = jnp.sum(t_ref[...], axis=0)
    m = jnp.max(t, axis=1, keepdims=True)
    o_ref[...] = jnp.sum(m, axis=0, keepdims=True) * inv_n


def _make_hist(n, n_clus, n_clas):
    info = plsc.get_sparse_core_info()
    nc, ns = 1, info.num_subcores
    nw = nc * ns
    chunk = n // nw
    unroll = 4
    assert chunk % (_LANES * unroll) == 0 and chunk % 8 == 0
    mesh = plsc.VectorSubcoreMesh(
        core_axis_name="c", subcore_axis_name="s", num_cores=nc)

    @functools.partial(
        pl.kernel,
        out_type=jax.ShapeDtypeStruct((nw, n_clus, n_clas), jnp.float32),
        mesh=mesh,
        scratch_types=[
            pltpu.VMEM((chunk,), jnp.float32),
            pltpu.VMEM((chunk,), jnp.int32),
            pltpu.VMEM((n_clus, n_clas), jnp.float32),
        ],
        compiler_params=pltpu.CompilerParams(needs_layout_passes=False),
    )
    def hist(clus_hbm, tgt_hbm, out_hbm, clus_v, tgt_v, tab):
        wid = lax.axis_index("s") * nc + lax.axis_index("c")
        base = wid * chunk
        pltpu.sync_copy(clus_hbm.at[pl.ds(base, chunk)], clus_v)
        pltpu.sync_copy(tgt_hbm.at[pl.ds(base, chunk)], tgt_v)

        zeros = jnp.zeros((_LANES,), jnp.float32)

        def zero_step(i, carry):
            for j in range(n_clas // _LANES):
                tab[i, pl.ds(j * _LANES, _LANES)] = zeros
            return carry

        lax.fori_loop(0, n_clus, zero_step, 0)

        ones = jnp.ones((_LANES,), jnp.float32)

        def step(i, carry):
            for j in range(unroll):
                off = (i * unroll + j) * _LANES
                cv = clus_v[pl.ds(off, _LANES)].astype(jnp.int32)
                tv = tgt_v[pl.ds(off, _LANES)]
                plsc.addupdate_scatter(tab, [cv, tv], ones)
            return carry

        lax.fori_loop(0, chunk // (_LANES * unroll), step, 0)
        pltpu.sync_copy(tab, out_hbm.at[wid])

    return hist, nw


def kernel(inputs, targets):
    n, n_clus = inputs.shape
    n_clas = 128
    row_blk = 64

    x3 = inputs.reshape(n // n_clus, n_clus, n_clus)
    lane_idx = jnp.asarray(
        np.arange(n_clus, dtype=np.float32).reshape(1, 1, n_clus))
    clus = pl.pallas_call(
        _argmax_body,
        grid=(x3.shape[0] // row_blk,),
        in_specs=[
            pl.BlockSpec((row_blk, n_clus, n_clus), lambda i: (i, 0, 0)),
            pl.BlockSpec((1, 1, n_clus), lambda i: (0, 0, 0)),
        ],
        out_specs=pl.BlockSpec((row_blk, n_clus), lambda i: (i, 0)),
        out_shape=jax.ShapeDtypeStruct((n // n_clus, n_clus), jnp.float32),
    )(x3, lane_idx)

    hist, nw = _make_hist(n, n_clus, n_clas)
    tables = hist(clus.reshape(n), targets)

    out = pl.pallas_call(
        functools.partial(_finish_body, inv_n=1.0 / n),
        out_shape=jax.ShapeDtypeStruct((1, 1), jnp.float32),
    )(tables)
    return out.reshape(1)
